# Initial kernel scaffold; baseline (speedup 1.0000x reference)
import jax
import jax.numpy as jnp
from jax.experimental import pallas as pl


def kernel(x, edge_index, edge_weights, W1, att_src1, att_dst1, bias1, We1, att_edge1, Wmu, att_src_mu, att_dst_mu, bias_mu, We_mu, att_edge_mu, Wls, att_src_ls, att_dst_ls, bias_ls):
    raise NotImplementedError("write your pallas kernel here")


# speedup vs baseline: 10.5313x; 10.5313x over previous
import functools

import jax
import jax.numpy as jnp
from jax import lax
from jax.experimental import pallas as pl
from jax.experimental.pallas import tpu as pltpu
from jax.experimental.pallas import tpu_sc as plsc

N = 10000
E = 320000
D = 128
H = 256
C = 128
ET = E + N
NW = 32
CHUNK = 10368
ET_PAD = NW * CHUNK
K = 48
NP = 10112
TROW = 632
ZR = 88

_f32 = jnp.float32
_i32 = jnp.int32


FH = 64


def _build_sc_edge():
  mesh = plsc.VectorSubcoreMesh(core_axis_name="c", subcore_axis_name="s",
                                num_cores=2, num_subcores=16)

  @functools.partial(
      pl.kernel,
      out_type=(
          jax.ShapeDtypeStruct((2, 2, NP, FH), _f32),
          jax.ShapeDtypeStruct((2 * NP,), _f32),
          jax.ShapeDtypeStruct((256,), _f32),
      ),
      mesh=mesh,
      compiler_params=pltpu.CompilerParams(needs_layout_passes=False,
                                           use_tc_tiling_on_sc=False),
      scratch_types=[
          pltpu.VMEM((NP,), _f32),
          pltpu.VMEM((NP,), _f32),
          pltpu.VMEM((CHUNK,), _i32),
          pltpu.VMEM((CHUNK,), _i32),
          pltpu.VMEM((CHUNK,), _f32),
          pltpu.VMEM((CHUNK,), _f32),
          pltpu.VMEM((K, FH), _f32),
          pltpu.VMEM((K,), _i32),
          pltpu.VMEM((K,), _i32),
          pltpu.VMEM((ZR, FH), _f32),
          pltpu.VMEM((640,), _f32),
          pltpu.VMEM((128,), _f32),
          pltpu.VMEM((2048,), _f32),
          pltpu.VMEM_SHARED((NP, FH), _f32),
          pltpu.VMEM_SHARED((NP,), _f32),
          pltpu.VMEM_SHARED((2048,), _f32),
          pltpu.SemaphoreType.DMA,
      ],
  )
  def sc_edge(h0_hbm, h1_hbm, as_hbm, ad_hbm, et_hbm, src_hbm, dst_hbm,
              acc_out, s_out, m_out,
              as_v, ad_v, src_v, dst_v, et_v, alpha_v, rows, sidx, didx,
              zbuf, zvec, mxr, maxbuf, acc_sh, svec_sh, maxsh, sem):
    c = lax.axis_index("c")
    s = lax.axis_index("s")
    wid = s * 2 + c
    base = wid * CHUNK

    pltpu.sync_copy(as_hbm, as_v)
    pltpu.sync_copy(ad_hbm, ad_v)
    pltpu.sync_copy(src_hbm.at[pl.ds(base, CHUNK)], src_v)
    pltpu.sync_copy(dst_hbm.at[pl.ds(base, CHUNK)], dst_v)
    pltpu.sync_copy(et_hbm.at[pl.ds(base, CHUNK)], et_v)

    zeros16 = jnp.zeros((16,), _f32)

    def zero_zbuf(r, carry):
      for f in range(FH // 16):
        zbuf[r, pl.ds(f * 16, 16)] = zeros16
      return carry

    lax.fori_loop(0, ZR, zero_zbuf, 0)
    for j in range(640 // 16):
      zvec[pl.ds(j * 16, 16)] = zeros16

    def zero_acc_slice():
      for kk in range(7):
        pltpu.sync_copy(zbuf, acc_sh.at[pl.ds(s * TROW + kk * ZR, ZR)])
      pltpu.sync_copy(zbuf.at[pl.ds(0, 16)],
                      acc_sh.at[pl.ds(s * TROW + 7 * ZR, 16)])

    zero_acc_slice()

    @pl.when(s == 0)
    def _zero_svec():
      for kk in range(15):
        pltpu.sync_copy(zvec, svec_sh.at[pl.ds(kk * 640, 640)])
      pltpu.sync_copy(zvec.at[pl.ds(0, 512)], svec_sh.at[pl.ds(9600, 512)])

    def pass_a(i, mx):
      sl = pl.ds(i * 16, 16)
      si = src_v[sl]
      di = dst_v[sl]
      a = (plsc.load_gather(as_v, [si]) + plsc.load_gather(ad_v, [di])
           + et_v[sl])
      a = jnp.where(a >= 0.0, a, 0.2 * a)
      alpha_v[sl] = a
      return jnp.maximum(mx, a)

    mx = lax.fori_loop(0, CHUNK // 16, pass_a,
                       jnp.full((16,), -3e38, _f32))
    for j in range(8):
      mxr[pl.ds(j * 16, 16)] = mx
    pltpu.sync_copy(mxr, maxsh.at[pl.ds(s * 128, 128)])
    plsc.subcore_barrier()
    pltpu.sync_copy(maxsh, maxbuf)

    def red_max(j, m):
      return jnp.maximum(m, maxbuf[pl.ds(j * 128, 16)])

    mall = lax.fori_loop(0, 16, red_max, jnp.full((16,), -3e38, _f32))
    mtot = jnp.max(mall)

    @pl.when(s == 0)
    def _write_m():
      mv16 = jnp.full((16,), mtot, _f32)
      for j in range(8):
        mxr[pl.ds(j * 16, 16)] = mv16
      pltpu.sync_copy(mxr, m_out.at[pl.ds(c * 128, 128)])

    def pass_b(i, carry):
      sl = pl.ds(i * 16, 16)
      alpha_v[sl] = jnp.exp(alpha_v[sl] - mtot)
      return carry

    lax.fori_loop(0, CHUNK // 16, pass_b, 0)

    for half, h_hbm in enumerate((h0_hbm, h1_hbm)):
      if half == 1:
        zero_acc_slice()
      plsc.subcore_barrier()

      def sc_chunk(ch, carry):
        b = ch * K
        for j in range(K // 16):
          sidx[pl.ds(j * 16, 16)] = src_v[pl.ds(b + j * 16, 16)]
          didx[pl.ds(j * 16, 16)] = dst_v[pl.ds(b + j * 16, 16)]
        pltpu.async_copy(h_hbm.at[sidx], rows, sem).wait()

        def scale_row(r, cc):
          e_b = plsc.load_gather(alpha_v, [jnp.full((16,), b + r, _i32)])
          for f in range(FH // 16):
            sl2 = pl.ds(f * 16, 16)
            rows[r, sl2] = rows[r, sl2] * e_b
          return cc

        lax.fori_loop(0, K, scale_row, 0)
        pltpu.sync_copy(rows, acc_sh.at[didx], add=True)
        if half == 0:
          pltpu.sync_copy(alpha_v.at[pl.ds(b, K)], svec_sh.at[didx],
                          add=True)
        return carry

      lax.fori_loop(0, CHUNK // K, sc_chunk, 0)
      plsc.subcore_barrier()

      pltpu.sync_copy(acc_sh.at[pl.ds(s * TROW, TROW)],
                      acc_out.at[c, half, pl.ds(s * TROW, TROW)])

    @pl.when(s == 0)
    def _write_s():
      pltpu.sync_copy(svec_sh, s_out.at[pl.ds(c * NP, NP)])

  return sc_edge


_SC_EDGE_CACHE = {}


def _sc_half_call(h0, h1, asv, adv, et, src, dst):
  if "k" not in _SC_EDGE_CACHE:
    _SC_EDGE_CACHE["k"] = _build_sc_edge()
  pad = jnp.zeros((NP - N,), _f32)
  return _SC_EDGE_CACHE["k"](
      h0, h1, jnp.concatenate([asv, pad]), jnp.concatenate([adv, pad]),
      et, src, dst)


def _sc_edge(h, asv, adv, et, src, dst):
  F = h.shape[1]
  accs = []
  sv = mv = None
  for q in range(F // (2 * FH)):
    h0 = h[:, q * 2 * FH:q * 2 * FH + FH]
    h1 = h[:, q * 2 * FH + FH:(q + 1) * 2 * FH]
    acc, svq, mvq = _sc_half_call(h0, h1, asv, adv, et, src, dst)
    accs.append(acc)
    if sv is None:
      sv, mv = svq, mvq
  acc_full = jnp.concatenate(
      [jnp.concatenate([a[:, 0, :N, :], a[:, 1, :N, :]], axis=2)
       for a in accs], axis=2)
  return (acc_full, jnp.stack([sv[:N], sv[NP:NP + N]]),
          jnp.stack([mv[:16], mv[128:144]]))


_BN = 1000


def _dense1_body(x_ref, w_ref, ats_ref, atd_ref, h_ref, as_ref, ad_ref):
  h = jnp.dot(x_ref[...], w_ref[...], preferred_element_type=_f32)
  h_ref[...] = h
  as_ref[...] = jnp.dot(h, ats_ref[...], preferred_element_type=_f32)
  ad_ref[...] = jnp.dot(h, atd_ref[...], preferred_element_type=_f32)


def _dense1(x, w, ats, atd):
  grid = (N // _BN,)
  return pl.pallas_call(
      _dense1_body,
      grid=grid,
      in_specs=[
          pl.BlockSpec((_BN, D), lambda i: (i, 0)),
          pl.BlockSpec((D, H), lambda i: (0, 0)),
          pl.BlockSpec((H, 1), lambda i: (0, 0)),
          pl.BlockSpec((H, 1), lambda i: (0, 0)),
      ],
      out_specs=[
          pl.BlockSpec((_BN, H), lambda i: (i, 0)),
          pl.BlockSpec((_BN, 1), lambda i: (i, 0)),
          pl.BlockSpec((_BN, 1), lambda i: (i, 0)),
      ],
      out_shape=[
          jax.ShapeDtypeStruct((N, H), _f32),
          jax.ShapeDtypeStruct((N, 1), _f32),
          jax.ShapeDtypeStruct((N, 1), _f32),
      ],
  )(x, w, ats, atd)


_BE = 8000


def _eterm_body(ew_ref, we_ref, ate_ref, et_ref, sum_ref):
  i = pl.program_id(0)
  w4 = jnp.dot(we_ref[...], ate_ref[...], preferred_element_type=_f32)
  et = jnp.dot(ew_ref[...], w4, preferred_element_type=_f32)
  et_ref[...] = et

  @pl.when(i == 0)
  def _():
    sum_ref[...] = jnp.zeros_like(sum_ref)

  sum_ref[...] += jnp.sum(et, axis=(0, 1), keepdims=True)


def _eterm(ew, we, ate):
  fh = we.shape[1]
  return pl.pallas_call(
      _eterm_body,
      grid=(E // _BE,),
      in_specs=[
          pl.BlockSpec((_BE, 4), lambda i: (i, 0)),
          pl.BlockSpec((4, fh), lambda i: (0, 0)),
          pl.BlockSpec((fh, 1), lambda i: (0, 0)),
      ],
      out_specs=[
          pl.BlockSpec((_BE, 1), lambda i: (i, 0)),
          pl.BlockSpec((1, 1), lambda i: (0, 0)),
      ],
      out_shape=[
          jax.ShapeDtypeStruct((E, 1), _f32),
          jax.ShapeDtypeStruct((1, 1), _f32),
      ],
  )(ew, we, ate)


def _combine_body(aa0_ref, aa1_ref, ab0_ref, ab1_ref, s0_ref, s1_ref,
                  w0_ref, w1_ref, b1_ref, wmu_ref, atsmu_ref, atdmu_ref,
                  wls_ref, atsls_ref, atdls_ref,
                  hmu_ref, asmu_ref, admu_ref, hls_ref, asls_ref, adls_ref):
  w0 = w0_ref[0, 0]
  w1 = w1_ref[0, 0]
  inv = 1.0 / (w0 * s0_ref[...] + w1 * s1_ref[...] + 1e-16)
  n0 = (w0 * aa0_ref[...] + w1 * aa1_ref[...]) * inv
  n1 = (w0 * ab0_ref[...] + w1 * ab1_ref[...]) * inv
  z = jnp.concatenate([n0, n1], axis=1) + b1_ref[...]
  z = jnp.maximum(z, 0.0)
  hmu = jnp.dot(z, wmu_ref[...], preferred_element_type=_f32)
  hmu_ref[...] = hmu
  asmu_ref[...] = jnp.dot(hmu, atsmu_ref[...], preferred_element_type=_f32)
  admu_ref[...] = jnp.dot(hmu, atdmu_ref[...], preferred_element_type=_f32)
  hls = jnp.dot(z, wls_ref[...], preferred_element_type=_f32)
  hls_ref[...] = hls
  asls_ref[...] = jnp.dot(hls, atsls_ref[...], preferred_element_type=_f32)
  adls_ref[...] = jnp.dot(hls, atdls_ref[...], preferred_element_type=_f32)


def _combine(aa0, aa1, ab0, ab1, s0, s1, w0, w1, b1, wmu, atsmu, atdmu,
             wls, atsls, atdls):
  blk = lambda r, cdim: pl.BlockSpec((_BN, cdim), lambda i: (i, 0))
  cst = lambda a, b: pl.BlockSpec((a, b), lambda i: (0, 0))
  return pl.pallas_call(
      _combine_body,
      grid=(N // _BN,),
      in_specs=[
          blk(_BN, C), blk(_BN, C), blk(_BN, C), blk(_BN, C),
          blk(_BN, 1), blk(_BN, 1),
          cst(1, 1), cst(1, 1), cst(1, H),
          cst(H, C), cst(C, 1), cst(C, 1),
          cst(H, C), cst(C, 1), cst(C, 1),
      ],
      out_specs=[
          blk(_BN, C), blk(_BN, 1), blk(_BN, 1),
          blk(_BN, C), blk(_BN, 1), blk(_BN, 1),
      ],
      out_shape=[
          jax.ShapeDtypeStruct((N, C), _f32),
          jax.ShapeDtypeStruct((N, 1), _f32),
          jax.ShapeDtypeStruct((N, 1), _f32),
          jax.ShapeDtypeStruct((N, C), _f32),
          jax.ShapeDtypeStruct((N, 1), _f32),
          jax.ShapeDtypeStruct((N, 1), _f32),
      ],
  )(aa0, aa1, ab0, ab1, s0, s1, w0, w1, b1, wmu, atsmu, atdmu,
    wls, atsls, atdls)


def _finish_body(amu0_ref, amu1_ref, smu0_ref, smu1_ref, wmu0_ref, wmu1_ref,
                 bmu_ref, als0_ref, als1_ref, sls0_ref, sls1_ref,
                 wls0_ref, wls1_ref, bls_ref, mu_ref, ls_ref):
  w0 = wmu0_ref[0, 0]
  w1 = wmu1_ref[0, 0]
  inv = 1.0 / (w0 * smu0_ref[...] + w1 * smu1_ref[...] + 1e-16)
  mu_ref[...] = (w0 * amu0_ref[...] + w1 * amu1_ref[...]) * inv + bmu_ref[...]
  v0 = wls0_ref[0, 0]
  v1 = wls1_ref[0, 0]
  inv2 = 1.0 / (v0 * sls0_ref[...] + v1 * sls1_ref[...] + 1e-16)
  ls_ref[...] = (v0 * als0_ref[...] + v1 * als1_ref[...]) * inv2 + bls_ref[...]


def _finish(amu0, amu1, smu0, smu1, wmu0, wmu1, bmu,
            als0, als1, sls0, sls1, wls0, wls1, bls):
  blk = lambda cdim: pl.BlockSpec((_BN, cdim), lambda i: (i, 0))
  cst = lambda a, b: pl.BlockSpec((a, b), lambda i: (0, 0))
  return pl.pallas_call(
      _finish_body,
      grid=(N // _BN,),
      in_specs=[
          blk(C), blk(C), blk(1), blk(1), cst(1, 1), cst(1, 1), cst(1, C),
          blk(C), blk(C), blk(1), blk(1), cst(1, 1), cst(1, 1), cst(1, C),
      ],
      out_specs=[blk(C), blk(C)],
      out_shape=[
          jax.ShapeDtypeStruct((N, C), _f32),
          jax.ShapeDtypeStruct((N, C), _f32),
      ],
  )(amu0, amu1, smu0, smu1, wmu0, wmu1, bmu,
    als0, als1, sls0, sls1, wls0, wls1, bls)


def kernel(x, edge_index, edge_weights, W1, att_src1, att_dst1, bias1, We1,
           att_edge1, Wmu, att_src_mu, att_dst_mu, bias_mu, We_mu,
           att_edge_mu, Wls, att_src_ls, att_dst_ls, bias_ls):
  pad = ET_PAD - ET
  loops = jnp.arange(N, dtype=jnp.int32)
  zpad = jnp.zeros((pad,), jnp.int32)
  src = jnp.concatenate([edge_index[0], loops, zpad])
  dst = jnp.concatenate([edge_index[1], loops, zpad])
  neg = jnp.full((pad,), -1e30, _f32)

  et1_e, esum1 = _eterm(edge_weights, We1, att_edge1[:, None])
  fill1 = esum1[0, 0] / E
  et1 = jnp.concatenate([et1_e[:, 0], jnp.full((N,), fill1, _f32), neg])

  etmu_e, esummu = _eterm(edge_weights, We_mu, att_edge_mu[:, None])
  fillmu = esummu[0, 0] / E
  etmu = jnp.concatenate([etmu_e[:, 0], jnp.full((N,), fillmu, _f32), neg])

  etls = jnp.concatenate([jnp.zeros((ET,), _f32), neg])

  h1, as1, ad1 = _dense1(x, W1, att_src1[:, None], att_dst1[:, None])

  acc_a, s1v, m1 = _sc_edge(h1[:, :C], as1[:, 0], ad1[:, 0], et1, src, dst)
  acc_b, _, _ = _sc_edge(h1[:, C:], as1[:, 0], ad1[:, 0], et1, src, dst)

  g1 = jnp.maximum(m1[0, 0], m1[1, 0])
  w10 = jnp.exp(m1[0, 0] - g1)[None, None]
  w11 = jnp.exp(m1[1, 0] - g1)[None, None]

  hmu, asmu, admu, hls, asls, adls = _combine(
      acc_a[0], acc_a[1], acc_b[0], acc_b[1],
      s1v[0][:, None], s1v[1][:, None], w10, w11, bias1[None, :],
      Wmu, att_src_mu[:, None], att_dst_mu[:, None],
      Wls, att_src_ls[:, None], att_dst_ls[:, None])

  accmu, smu, mmu = _sc_edge(hmu, asmu[:, 0], admu[:, 0], etmu, src, dst)
  accls, sls, mls = _sc_edge(hls, asls[:, 0], adls[:, 0], etls, src, dst)

  gmu = jnp.maximum(mmu[0, 0], mmu[1, 0])
  wmu0 = jnp.exp(mmu[0, 0] - gmu)[None, None]
  wmu1 = jnp.exp(mmu[1, 0] - gmu)[None, None]
  gls = jnp.maximum(mls[0, 0], mls[1, 0])
  wls0 = jnp.exp(mls[0, 0] - gls)[None, None]
  wls1 = jnp.exp(mls[1, 0] - gls)[None, None]

  mu, logstd = _finish(
      accmu[0], accmu[1], smu[0][:, None], smu[1][:, None], wmu0, wmu1,
      bias_mu[None, :],
      accls[0], accls[1], sls[0][:, None], sls[1][:, None], wls0, wls1,
      bias_ls[None, :])
  return (mu, logstd)


# speedup vs baseline: 17.0580x; 1.6197x over previous
import functools

import jax
import jax.numpy as jnp
from jax import lax
from jax.experimental import pallas as pl
from jax.experimental.pallas import tpu as pltpu
from jax.experimental.pallas import tpu_sc as plsc

N = 10000
E = 320000
D = 128
H = 256
C = 128
ET = E + N
NW = 32
CHUNK = 10368
ET_PAD = NW * CHUNK
K = 48
NP = 10112
TROW = 632
ZR = 88

_f32 = jnp.float32
_i32 = jnp.int32


FH = 64


def _build_sc_edge():
  mesh = plsc.VectorSubcoreMesh(core_axis_name="c", subcore_axis_name="s",
                                num_cores=2, num_subcores=16)

  @functools.partial(
      pl.kernel,
      out_type=(
          jax.ShapeDtypeStruct((2, 2, NP, FH), _f32),
          jax.ShapeDtypeStruct((2 * NP,), _f32),
          jax.ShapeDtypeStruct((256,), _f32),
      ),
      mesh=mesh,
      compiler_params=pltpu.CompilerParams(needs_layout_passes=False,
                                           use_tc_tiling_on_sc=False),
      scratch_types=[
          pltpu.VMEM((NP,), _f32),
          pltpu.VMEM((NP,), _f32),
          pltpu.VMEM((CHUNK,), _i32),
          pltpu.VMEM((CHUNK,), _i32),
          pltpu.VMEM((CHUNK,), _f32),
          pltpu.VMEM((CHUNK,), _f32),
          pltpu.VMEM((K, FH), _f32),
          pltpu.VMEM((K, FH), _f32),
          pltpu.VMEM((K, FH), _f32),
          pltpu.VMEM((K, FH), _f32),
          pltpu.VMEM((K,), _i32),
          pltpu.VMEM((K,), _i32),
          pltpu.VMEM((K,), _i32),
          pltpu.VMEM((K,), _i32),
          pltpu.VMEM((ZR, FH), _f32),
          pltpu.VMEM((640,), _f32),
          pltpu.VMEM((128,), _f32),
          pltpu.VMEM((2048,), _f32),
          pltpu.VMEM_SHARED((NP, FH), _f32),
          pltpu.VMEM_SHARED((NP,), _f32),
          pltpu.VMEM_SHARED((2048,), _f32),
          pltpu.SemaphoreType.DMA,
          pltpu.SemaphoreType.DMA,
      ],
  )
  def sc_edge(h0_hbm, h1_hbm, as_hbm, ad_hbm, et_hbm, src_hbm, dst_hbm,
              acc_out, s_out, m_out,
              as_v, ad_v, src_v, dst_v, et_v, alpha_v,
              gbuf0, gbuf1, sbuf0, sbuf1, sidx0, sidx1, didx0, didx1,
              zbuf, zvec, mxr, maxbuf, acc_sh, svec_sh, maxsh,
              gsem0, gsem1):
    c = lax.axis_index("c")
    s = lax.axis_index("s")
    wid = s * 2 + c
    base = wid * CHUNK
    gbuf = (gbuf0, gbuf1)
    sbuf = (sbuf0, sbuf1)
    sidx = (sidx0, sidx1)
    didx = (didx0, didx1)
    gsem = (gsem0, gsem1)
    NCH = CHUNK // K

    pltpu.sync_copy(as_hbm, as_v)
    pltpu.sync_copy(ad_hbm, ad_v)
    pltpu.sync_copy(src_hbm.at[pl.ds(base, CHUNK)], src_v)
    pltpu.sync_copy(dst_hbm.at[pl.ds(base, CHUNK)], dst_v)
    pltpu.sync_copy(et_hbm.at[pl.ds(base, CHUNK)], et_v)

    zeros16 = jnp.zeros((16,), _f32)

    def zero_zbuf(r, carry):
      for f in range(FH // 16):
        zbuf[r, pl.ds(f * 16, 16)] = zeros16
      return carry

    lax.fori_loop(0, ZR, zero_zbuf, 0)
    for j in range(640 // 16):
      zvec[pl.ds(j * 16, 16)] = zeros16

    def zero_acc_slice():
      for kk in range(7):
        pltpu.sync_copy(zbuf, acc_sh.at[pl.ds(s * TROW + kk * ZR, ZR)])
      pltpu.sync_copy(zbuf.at[pl.ds(0, 16)],
                      acc_sh.at[pl.ds(s * TROW + 7 * ZR, 16)])

    zero_acc_slice()

    @pl.when(s == 0)
    def _zero_svec():
      for kk in range(15):
        pltpu.sync_copy(zvec, svec_sh.at[pl.ds(kk * 640, 640)])
      pltpu.sync_copy(zvec.at[pl.ds(0, 512)], svec_sh.at[pl.ds(9600, 512)])

    def pass_a(i, mx):
      sl = pl.ds(i * 16, 16)
      si = src_v[sl]
      di = dst_v[sl]
      a = (plsc.load_gather(as_v, [si]) + plsc.load_gather(ad_v, [di])
           + et_v[sl])
      a = jnp.where(a >= 0.0, a, 0.2 * a)
      alpha_v[sl] = a
      return jnp.maximum(mx, a)

    mx = lax.fori_loop(0, CHUNK // 16, pass_a,
                       jnp.full((16,), -3e38, _f32))
    for j in range(8):
      mxr[pl.ds(j * 16, 16)] = mx
    pltpu.sync_copy(mxr, maxsh.at[pl.ds(s * 128, 128)])
    plsc.subcore_barrier()
    pltpu.sync_copy(maxsh, maxbuf)

    def red_max(j, m):
      return jnp.maximum(m, maxbuf[pl.ds(j * 128, 16)])

    mall = lax.fori_loop(0, 16, red_max, jnp.full((16,), -3e38, _f32))
    mtot = jnp.max(mall)

    @pl.when(s == 0)
    def _write_m():
      mv16 = jnp.full((16,), mtot, _f32)
      for j in range(8):
        mxr[pl.ds(j * 16, 16)] = mv16
      pltpu.sync_copy(mxr, m_out.at[pl.ds(c * 128, 128)])

    def pass_b(i, carry):
      sl = pl.ds(i * 16, 16)
      alpha_v[sl] = jnp.exp(alpha_v[sl] - mtot)
      return carry

    lax.fori_loop(0, CHUNK // 16, pass_b, 0)

    for half, h_hbm in enumerate((h0_hbm, h1_hbm)):
      if half == 1:
        zero_acc_slice()
      plsc.subcore_barrier()

      for j in range(2):
        for q in range(K // 16):
          sidx[j][pl.ds(q * 16, 16)] = src_v[pl.ds(j * K + q * 16, 16)]
        pltpu.async_copy(h_hbm.at[sidx[j]], gbuf[j], gsem[j])

      def group(g, carry):
        for j in range(2):
          i = 2 * g + j
          b = i * K
          pltpu.make_async_copy(h_hbm.at[sidx[j]], gbuf[j], gsem[j]).wait()

          for q in range(K // 16):
            didx[j][pl.ds(q * 16, 16)] = dst_v[pl.ds(b + q * 16, 16)]

          def scale_row(r, cc):
            e_b = plsc.load_gather(alpha_v, [jnp.full((16,), b + r, _i32)])
            for f in range(FH // 16):
              sl2 = pl.ds(f * 16, 16)
              sbuf[j][r, sl2] = gbuf[j][r, sl2] * e_b
            return cc

          lax.fori_loop(0, K, scale_row, 0)

          @pl.when(i + 2 < NCH)
          def _next():
            for q in range(K // 16):
              sidx[j][pl.ds(q * 16, 16)] = src_v[pl.ds(b + 2 * K + q * 16,
                                                       16)]
            pltpu.async_copy(h_hbm.at[sidx[j]], gbuf[j], gsem[j])

          pltpu.sync_copy(sbuf[j], acc_sh.at[didx[j]], add=True)
          if half == 0:
            pltpu.sync_copy(alpha_v.at[pl.ds(b, K)], svec_sh.at[didx[j]],
                            add=True)
        return carry

      lax.fori_loop(0, NCH // 2, group, 0)
      plsc.subcore_barrier()

      pltpu.sync_copy(acc_sh.at[pl.ds(s * TROW, TROW)],
                      acc_out.at[c, half, pl.ds(s * TROW, TROW)])

    @pl.when(s == 0)
    def _write_s():
      pltpu.sync_copy(svec_sh, s_out.at[pl.ds(c * NP, NP)])

  return sc_edge


_SC_EDGE_CACHE = {}


def _sc_half_call(h0, h1, asv, adv, et, src, dst):
  if "k" not in _SC_EDGE_CACHE:
    _SC_EDGE_CACHE["k"] = _build_sc_edge()
  pad = jnp.zeros((NP - N,), _f32)
  return _SC_EDGE_CACHE["k"](
      h0, h1, jnp.concatenate([asv, pad]), jnp.concatenate([adv, pad]),
      et, src, dst)


def _sc_edge(h, asv, adv, et, src, dst):
  F = h.shape[1]
  accs = []
  sv = mv = None
  for q in range(F // (2 * FH)):
    h0 = h[:, q * 2 * FH:q * 2 * FH + FH]
    h1 = h[:, q * 2 * FH + FH:(q + 1) * 2 * FH]
    acc, svq, mvq = _sc_half_call(h0, h1, asv, adv, et, src, dst)
    accs.append(acc)
    if sv is None:
      sv = jnp.stack([svq[:N], svq[NP:NP + N]])
      mv = mvq
  acc_full = jnp.concatenate(
      [jnp.concatenate([a[:, 0, :N, :FH], a[:, 1, :N, :FH]], axis=2)
       for a in accs], axis=2)
  return (acc_full, sv, jnp.stack([mv[:16], mv[128:144]]))


_BN = 1000


def _dense1_body(x_ref, w_ref, ats_ref, atd_ref, h_ref, as_ref, ad_ref):
  h = jnp.dot(x_ref[...], w_ref[...], preferred_element_type=_f32)
  h_ref[...] = h
  as_ref[...] = jnp.dot(h, ats_ref[...], preferred_element_type=_f32)
  ad_ref[...] = jnp.dot(h, atd_ref[...], preferred_element_type=_f32)


def _dense1(x, w, ats, atd):
  grid = (N // _BN,)
  return pl.pallas_call(
      _dense1_body,
      grid=grid,
      in_specs=[
          pl.BlockSpec((_BN, D), lambda i: (i, 0)),
          pl.BlockSpec((D, H), lambda i: (0, 0)),
          pl.BlockSpec((H, 1), lambda i: (0, 0)),
          pl.BlockSpec((H, 1), lambda i: (0, 0)),
      ],
      out_specs=[
          pl.BlockSpec((_BN, H), lambda i: (i, 0)),
          pl.BlockSpec((_BN, 1), lambda i: (i, 0)),
          pl.BlockSpec((_BN, 1), lambda i: (i, 0)),
      ],
      out_shape=[
          jax.ShapeDtypeStruct((N, H), _f32),
          jax.ShapeDtypeStruct((N, 1), _f32),
          jax.ShapeDtypeStruct((N, 1), _f32),
      ],
  )(x, w, ats, atd)


_BE = 8000


def _eterm_body(ew_ref, we_ref, ate_ref, et_ref, sum_ref):
  i = pl.program_id(0)
  w4 = jnp.dot(we_ref[...], ate_ref[...], preferred_element_type=_f32)
  et = jnp.dot(ew_ref[...], w4, preferred_element_type=_f32)
  et_ref[...] = et

  @pl.when(i == 0)
  def _():
    sum_ref[...] = jnp.zeros_like(sum_ref)

  sum_ref[...] += jnp.sum(et, axis=(0, 1), keepdims=True)


def _eterm(ew, we, ate):
  fh = we.shape[1]
  return pl.pallas_call(
      _eterm_body,
      grid=(E // _BE,),
      in_specs=[
          pl.BlockSpec((_BE, 4), lambda i: (i, 0)),
          pl.BlockSpec((4, fh), lambda i: (0, 0)),
          pl.BlockSpec((fh, 1), lambda i: (0, 0)),
      ],
      out_specs=[
          pl.BlockSpec((_BE, 1), lambda i: (i, 0)),
          pl.BlockSpec((1, 1), lambda i: (0, 0)),
      ],
      out_shape=[
          jax.ShapeDtypeStruct((E, 1), _f32),
          jax.ShapeDtypeStruct((1, 1), _f32),
      ],
  )(ew, we, ate)


def _combine_body(aa0_ref, aa1_ref, ab0_ref, ab1_ref, s0_ref, s1_ref,
                  w0_ref, w1_ref, b1_ref, wmu_ref, atsmu_ref, atdmu_ref,
                  wls_ref, atsls_ref, atdls_ref,
                  hmu_ref, asmu_ref, admu_ref, hls_ref, asls_ref, adls_ref):
  w0 = w0_ref[0, 0]
  w1 = w1_ref[0, 0]
  inv = 1.0 / (w0 * s0_ref[...] + w1 * s1_ref[...] + 1e-16)
  n0 = (w0 * aa0_ref[...] + w1 * aa1_ref[...]) * inv
  n1 = (w0 * ab0_ref[...] + w1 * ab1_ref[...]) * inv
  z = jnp.concatenate([n0, n1], axis=1) + b1_ref[...]
  z = jnp.maximum(z, 0.0)
  hmu = jnp.dot(z, wmu_ref[...], preferred_element_type=_f32)
  hmu_ref[...] = hmu
  asmu_ref[...] = jnp.dot(hmu, atsmu_ref[...], preferred_element_type=_f32)
  admu_ref[...] = jnp.dot(hmu, atdmu_ref[...], preferred_element_type=_f32)
  hls = jnp.dot(z, wls_ref[...], preferred_element_type=_f32)
  hls_ref[...] = hls
  asls_ref[...] = jnp.dot(hls, atsls_ref[...], preferred_element_type=_f32)
  adls_ref[...] = jnp.dot(hls, atdls_ref[...], preferred_element_type=_f32)


def _combine(aa0, aa1, ab0, ab1, s0, s1, w0, w1, b1, wmu, atsmu, atdmu,
             wls, atsls, atdls):
  blk = lambda r, cdim: pl.BlockSpec((_BN, cdim), lambda i: (i, 0))
  cst = lambda a, b: pl.BlockSpec((a, b), lambda i: (0, 0))
  return pl.pallas_call(
      _combine_body,
      grid=(N // _BN,),
      in_specs=[
          blk(_BN, C), blk(_BN, C), blk(_BN, C), blk(_BN, C),
          blk(_BN, 1), blk(_BN, 1),
          cst(1, 1), cst(1, 1), cst(1, H),
          cst(H, C), cst(C, 1), cst(C, 1),
          cst(H, C), cst(C, 1), cst(C, 1),
      ],
      out_specs=[
          blk(_BN, C), blk(_BN, 1), blk(_BN, 1),
          blk(_BN, C), blk(_BN, 1), blk(_BN, 1),
      ],
      out_shape=[
          jax.ShapeDtypeStruct((N, C), _f32),
          jax.ShapeDtypeStruct((N, 1), _f32),
          jax.ShapeDtypeStruct((N, 1), _f32),
          jax.ShapeDtypeStruct((N, C), _f32),
          jax.ShapeDtypeStruct((N, 1), _f32),
          jax.ShapeDtypeStruct((N, 1), _f32),
      ],
  )(aa0, aa1, ab0, ab1, s0, s1, w0, w1, b1, wmu, atsmu, atdmu,
    wls, atsls, atdls)


def _finish_body(amu0_ref, amu1_ref, smu0_ref, smu1_ref, wmu0_ref, wmu1_ref,
                 bmu_ref, als0_ref, als1_ref, sls0_ref, sls1_ref,
                 wls0_ref, wls1_ref, bls_ref, mu_ref, ls_ref):
  w0 = wmu0_ref[0, 0]
  w1 = wmu1_ref[0, 0]
  inv = 1.0 / (w0 * smu0_ref[...] + w1 * smu1_ref[...] + 1e-16)
  mu_ref[...] = (w0 * amu0_ref[...] + w1 * amu1_ref[...]) * inv + bmu_ref[...]
  v0 = wls0_ref[0, 0]
  v1 = wls1_ref[0, 0]
  inv2 = 1.0 / (v0 * sls0_ref[...] + v1 * sls1_ref[...] + 1e-16)
  ls_ref[...] = (v0 * als0_ref[...] + v1 * als1_ref[...]) * inv2 + bls_ref[...]


def _finish(amu0, amu1, smu0, smu1, wmu0, wmu1, bmu,
            als0, als1, sls0, sls1, wls0, wls1, bls):
  blk = lambda cdim: pl.BlockSpec((_BN, cdim), lambda i: (i, 0))
  cst = lambda a, b: pl.BlockSpec((a, b), lambda i: (0, 0))
  return pl.pallas_call(
      _finish_body,
      grid=(N // _BN,),
      in_specs=[
          blk(C), blk(C), blk(1), blk(1), cst(1, 1), cst(1, 1), cst(1, C),
          blk(C), blk(C), blk(1), blk(1), cst(1, 1), cst(1, 1), cst(1, C),
      ],
      out_specs=[blk(C), blk(C)],
      out_shape=[
          jax.ShapeDtypeStruct((N, C), _f32),
          jax.ShapeDtypeStruct((N, C), _f32),
      ],
  )(amu0, amu1, smu0, smu1, wmu0, wmu1, bmu,
    als0, als1, sls0, sls1, wls0, wls1, bls)


def kernel(x, edge_index, edge_weights, W1, att_src1, att_dst1, bias1, We1,
           att_edge1, Wmu, att_src_mu, att_dst_mu, bias_mu, We_mu,
           att_edge_mu, Wls, att_src_ls, att_dst_ls, bias_ls):
  pad = ET_PAD - ET
  loops = jnp.arange(N, dtype=jnp.int32)
  zpad = jnp.zeros((pad,), jnp.int32)
  src = jnp.concatenate([edge_index[0], loops, zpad])
  dst = jnp.concatenate([edge_index[1], loops, zpad])
  neg = jnp.full((pad,), -1e30, _f32)

  et1_e, esum1 = _eterm(edge_weights, We1, att_edge1[:, None])
  fill1 = esum1[0, 0] / E
  et1 = jnp.concatenate([et1_e[:, 0], jnp.full((N,), fill1, _f32), neg])

  etmu_e, esummu = _eterm(edge_weights, We_mu, att_edge_mu[:, None])
  fillmu = esummu[0, 0] / E
  etmu = jnp.concatenate([etmu_e[:, 0], jnp.full((N,), fillmu, _f32), neg])

  etls = jnp.concatenate([jnp.zeros((ET,), _f32), neg])

  h1, as1, ad1 = _dense1(x, W1, att_src1[:, None], att_dst1[:, None])

  acc_a, s1v, m1 = _sc_edge(h1[:, :C], as1[:, 0], ad1[:, 0], et1, src, dst)
  acc_b, _, _ = _sc_edge(h1[:, C:], as1[:, 0], ad1[:, 0], et1, src, dst)

  g1 = jnp.maximum(m1[0, 0], m1[1, 0])
  w10 = jnp.exp(m1[0, 0] - g1)[None, None]
  w11 = jnp.exp(m1[1, 0] - g1)[None, None]

  hmu, asmu, admu, hls, asls, adls = _combine(
      acc_a[0], acc_a[1], acc_b[0], acc_b[1],
      s1v[0][:, None], s1v[1][:, None], w10, w11, bias1[None, :],
      Wmu, att_src_mu[:, None], att_dst_mu[:, None],
      Wls, att_src_ls[:, None], att_dst_ls[:, None])

  accmu, smu, mmu = _sc_edge(hmu, asmu[:, 0], admu[:, 0], etmu, src, dst)
  accls, sls, mls = _sc_edge(hls, asls[:, 0], adls[:, 0], etls, src, dst)

  gmu = jnp.maximum(mmu[0, 0], mmu[1, 0])
  wmu0 = jnp.exp(mmu[0, 0] - gmu)[None, None]
  wmu1 = jnp.exp(mmu[1, 0] - gmu)[None, None]
  gls = jnp.maximum(mls[0, 0], mls[1, 0])
  wls0 = jnp.exp(mls[0, 0] - gls)[None, None]
  wls1 = jnp.exp(mls[1, 0] - gls)[None, None]

  mu, logstd = _finish(
      accmu[0], accmu[1], smu[0][:, None], smu[1][:, None], wmu0, wmu1,
      bias_mu[None, :],
      accls[0], accls[1], sls[0][:, None], sls[1][:, None], wls0, wls1,
      bias_ls[None, :])
  return (mu, logstd)


# speedup vs baseline: 17.4454x; 1.0227x over previous
import functools

import jax
import jax.numpy as jnp
from jax import lax
from jax.experimental import pallas as pl
from jax.experimental.pallas import tpu as pltpu
from jax.experimental.pallas import tpu_sc as plsc

N = 10000
E = 320000
D = 128
H = 256
C = 128
ET = E + N
NW = 32
CHUNK = 10368
ET_PAD = NW * CHUNK
K = 64
NP = 10112
TROW = 632
ZR = 88

_f32 = jnp.float32
_i32 = jnp.int32


FH = 64


def _build_sc_edge():
  mesh = plsc.VectorSubcoreMesh(core_axis_name="c", subcore_axis_name="s",
                                num_cores=2, num_subcores=16)

  @functools.partial(
      pl.kernel,
      out_type=(
          jax.ShapeDtypeStruct((2, 2, NP, FH), _f32),
          jax.ShapeDtypeStruct((2 * NP,), _f32),
          jax.ShapeDtypeStruct((256,), _f32),
      ),
      mesh=mesh,
      compiler_params=pltpu.CompilerParams(needs_layout_passes=False,
                                           use_tc_tiling_on_sc=False),
      scratch_types=[
          pltpu.VMEM((NP,), _f32),
          pltpu.VMEM((NP,), _f32),
          pltpu.VMEM((CHUNK,), _i32),
          pltpu.VMEM((CHUNK,), _i32),
          pltpu.VMEM((CHUNK,), _f32),
          pltpu.VMEM((CHUNK,), _f32),
          pltpu.VMEM((K, FH), _f32),
          pltpu.VMEM((K, FH), _f32),
          pltpu.VMEM((K, FH), _f32),
          pltpu.VMEM((K, FH), _f32),
          pltpu.VMEM((K,), _i32),
          pltpu.VMEM((K,), _i32),
          pltpu.VMEM((K,), _i32),
          pltpu.VMEM((K,), _i32),
          pltpu.VMEM((2 * K,), _i32),
          pltpu.VMEM((ZR, FH), _f32),
          pltpu.VMEM((640,), _f32),
          pltpu.VMEM((128,), _f32),
          pltpu.VMEM((2048,), _f32),
          pltpu.VMEM_SHARED((NP, FH), _f32),
          pltpu.VMEM_SHARED((NP,), _f32),
          pltpu.VMEM_SHARED((2048,), _f32),
          pltpu.SemaphoreType.DMA,
          pltpu.SemaphoreType.DMA,
      ],
  )
  def sc_edge(h0_hbm, h1_hbm, as_hbm, ad_hbm, et_hbm, src_hbm, dst_hbm,
              acc_out, s_out, m_out,
              as_v, ad_v, src_v, dst_v, et_v, alpha_v,
              gbuf0, gbuf1, sbuf0, sbuf1, sidx0, sidx1, didx0, didx1,
              vidx, zbuf, zvec, mxr, maxbuf, acc_sh, svec_sh, maxsh,
              gsem0, gsem1):
    c = lax.axis_index("c")
    s = lax.axis_index("s")
    wid = s * 2 + c
    base = wid * CHUNK
    gbuf = (gbuf0, gbuf1)
    sbuf = (sbuf0, sbuf1)
    sidx = (sidx0, sidx1)
    didx = (didx0, didx1)
    gsem = (gsem0, gsem1)
    NCH = CHUNK // K

    pltpu.sync_copy(as_hbm, as_v)
    pltpu.sync_copy(ad_hbm, ad_v)
    pltpu.sync_copy(src_hbm.at[pl.ds(base, CHUNK)], src_v)
    pltpu.sync_copy(dst_hbm.at[pl.ds(base, CHUNK)], dst_v)
    pltpu.sync_copy(et_hbm.at[pl.ds(base, CHUNK)], et_v)

    zeros16 = jnp.zeros((16,), _f32)

    def zero_zbuf(r, carry):
      for f in range(FH // 16):
        zbuf[r, pl.ds(f * 16, 16)] = zeros16
      return carry

    lax.fori_loop(0, ZR, zero_zbuf, 0)
    for j in range(640 // 16):
      zvec[pl.ds(j * 16, 16)] = zeros16

    def zero_acc_slice():
      for kk in range(7):
        pltpu.sync_copy(zbuf, acc_sh.at[pl.ds(s * TROW + kk * ZR, ZR)])
      pltpu.sync_copy(zbuf.at[pl.ds(0, 16)],
                      acc_sh.at[pl.ds(s * TROW + 7 * ZR, 16)])

    zero_acc_slice()

    @pl.when(s == 0)
    def _zero_svec():
      for kk in range(15):
        pltpu.sync_copy(zvec, svec_sh.at[pl.ds(kk * 640, 640)])
      pltpu.sync_copy(zvec.at[pl.ds(0, 512)], svec_sh.at[pl.ds(9600, 512)])

    def pass_a(i, mx):
      sl = pl.ds(i * 16, 16)
      si = src_v[sl]
      di = dst_v[sl]
      a = (plsc.load_gather(as_v, [si]) + plsc.load_gather(ad_v, [di])
           + et_v[sl])
      a = jnp.where(a >= 0.0, a, 0.2 * a)
      alpha_v[sl] = a
      return jnp.maximum(mx, a)

    mx = lax.fori_loop(0, CHUNK // 16, pass_a,
                       jnp.full((16,), -3e38, _f32))
    for j in range(8):
      mxr[pl.ds(j * 16, 16)] = mx
    pltpu.sync_copy(mxr, maxsh.at[pl.ds(s * 128, 128)])
    plsc.subcore_barrier()
    pltpu.sync_copy(maxsh, maxbuf)

    def red_max(j, m):
      return jnp.maximum(m, maxbuf[pl.ds(j * 128, 16)])

    mall = lax.fori_loop(0, 16, red_max, jnp.full((16,), -3e38, _f32))
    mtot = jnp.max(mall)

    @pl.when(s == 0)
    def _write_m():
      mv16 = jnp.full((16,), mtot, _f32)
      for j in range(8):
        mxr[pl.ds(j * 16, 16)] = mv16
      pltpu.sync_copy(mxr, m_out.at[pl.ds(c * 128, 128)])

    def pass_b(i, carry):
      sl = pl.ds(i * 16, 16)
      alpha_v[sl] = jnp.exp(alpha_v[sl] - mtot)
      return carry

    lax.fori_loop(0, CHUNK // 16, pass_b, 0)

    for half, h_hbm in enumerate((h0_hbm, h1_hbm)):
      if half == 1:
        zero_acc_slice()
      plsc.subcore_barrier()

      for j in range(2):
        for q in range(K // 16):
          sidx[j][pl.ds(q * 16, 16)] = src_v[pl.ds(j * K + q * 16, 16)]
        pltpu.async_copy(h_hbm.at[sidx[j]], gbuf[j], gsem[j])

      def group(g, carry):
        for j in range(2):
          i = 2 * g + j
          b = i * K
          pltpu.make_async_copy(h_hbm.at[sidx[j]], gbuf[j], gsem[j]).wait()

          for q in range(K // 16):
            didx[j][pl.ds(q * 16, 16)] = dst_v[pl.ds(b + q * 16, 16)]

          def scale_row(r, cc):
            e_b = plsc.load_gather(alpha_v, [jnp.full((16,), b + r, _i32)])
            for f in range(FH // 16):
              sl2 = pl.ds(f * 16, 16)
              sbuf[j][r, sl2] = gbuf[j][r, sl2] * e_b
            return cc

          lax.fori_loop(0, K, scale_row, 0)

          @pl.when(i + 2 < NCH)
          def _next():
            for q in range(K // 16):
              sidx[j][pl.ds(q * 16, 16)] = src_v[pl.ds(b + 2 * K + q * 16,
                                                       16)]
            pltpu.async_copy(h_hbm.at[sidx[j]], gbuf[j], gsem[j])

          pltpu.sync_copy(sbuf[j], acc_sh.at[didx[j]], add=True)
          if half == 0 and j == 1:
            for q in range(2 * K // 16):
              vidx[pl.ds(q * 16, 16)] = dst_v[pl.ds(b - K + q * 16, 16)]
            pltpu.sync_copy(alpha_v.at[pl.ds(b - K, 2 * K)],
                            svec_sh.at[vidx], add=True)
        return carry

      lax.fori_loop(0, NCH // 2, group, 0)
      plsc.subcore_barrier()

      pltpu.sync_copy(acc_sh.at[pl.ds(s * TROW, TROW)],
                      acc_out.at[c, half, pl.ds(s * TROW, TROW)])

    @pl.when(s == 0)
    def _write_s():
      pltpu.sync_copy(svec_sh, s_out.at[pl.ds(c * NP, NP)])

  return sc_edge


_SC_EDGE_CACHE = {}


def _sc_half_call(h0, h1, asv, adv, et, src, dst):
  if "k" not in _SC_EDGE_CACHE:
    _SC_EDGE_CACHE["k"] = _build_sc_edge()
  pad = jnp.zeros((NP - N,), _f32)
  return _SC_EDGE_CACHE["k"](
      h0, h1, jnp.concatenate([asv, pad]), jnp.concatenate([adv, pad]),
      et, src, dst)


def _sc_edge(h, asv, adv, et, src, dst):
  F = h.shape[1]
  accs = []
  sv = mv = None
  for q in range(F // (2 * FH)):
    h0 = h[:, q * 2 * FH:q * 2 * FH + FH]
    h1 = h[:, q * 2 * FH + FH:(q + 1) * 2 * FH]
    acc, svq, mvq = _sc_half_call(h0, h1, asv, adv, et, src, dst)
    accs.append(acc)
    if sv is None:
      sv = jnp.stack([svq[:N], svq[NP:NP + N]])
      mv = mvq
  acc_full = jnp.concatenate(
      [jnp.concatenate([a[:, 0, :N, :FH], a[:, 1, :N, :FH]], axis=2)
       for a in accs], axis=2)
  return (acc_full, sv, jnp.stack([mv[:16], mv[128:144]]))


_BN = 1000


def _dense1_body(x_ref, w_ref, ats_ref, atd_ref, h_ref, as_ref, ad_ref):
  h = jnp.dot(x_ref[...], w_ref[...], preferred_element_type=_f32)
  h_ref[...] = h
  as_ref[...] = jnp.dot(h, ats_ref[...], preferred_element_type=_f32)
  ad_ref[...] = jnp.dot(h, atd_ref[...], preferred_element_type=_f32)


def _dense1(x, w, ats, atd):
  grid = (N // _BN,)
  return pl.pallas_call(
      _dense1_body,
      grid=grid,
      in_specs=[
          pl.BlockSpec((_BN, D), lambda i: (i, 0)),
          pl.BlockSpec((D, H), lambda i: (0, 0)),
          pl.BlockSpec((H, 1), lambda i: (0, 0)),
          pl.BlockSpec((H, 1), lambda i: (0, 0)),
      ],
      out_specs=[
          pl.BlockSpec((_BN, H), lambda i: (i, 0)),
          pl.BlockSpec((_BN, 1), lambda i: (i, 0)),
          pl.BlockSpec((_BN, 1), lambda i: (i, 0)),
      ],
      out_shape=[
          jax.ShapeDtypeStruct((N, H), _f32),
          jax.ShapeDtypeStruct((N, 1), _f32),
          jax.ShapeDtypeStruct((N, 1), _f32),
      ],
  )(x, w, ats, atd)


_BE = 8000


def _eterm_body(ew_ref, we_ref, ate_ref, et_ref, sum_ref):
  i = pl.program_id(0)
  w4 = jnp.dot(we_ref[...], ate_ref[...], preferred_element_type=_f32)
  et = jnp.dot(ew_ref[...], w4, preferred_element_type=_f32)
  et_ref[...] = et

  @pl.when(i == 0)
  def _():
    sum_ref[...] = jnp.zeros_like(sum_ref)

  sum_ref[...] += jnp.sum(et, axis=(0, 1), keepdims=True)


def _eterm(ew, we, ate):
  fh = we.shape[1]
  return pl.pallas_call(
      _eterm_body,
      grid=(E // _BE,),
      in_specs=[
          pl.BlockSpec((_BE, 4), lambda i: (i, 0)),
          pl.BlockSpec((4, fh), lambda i: (0, 0)),
          pl.BlockSpec((fh, 1), lambda i: (0, 0)),
      ],
      out_specs=[
          pl.BlockSpec((_BE, 1), lambda i: (i, 0)),
          pl.BlockSpec((1, 1), lambda i: (0, 0)),
      ],
      out_shape=[
          jax.ShapeDtypeStruct((E, 1), _f32),
          jax.ShapeDtypeStruct((1, 1), _f32),
      ],
  )(ew, we, ate)


def _combine_body(aa0_ref, aa1_ref, ab0_ref, ab1_ref, s0_ref, s1_ref,
                  w0_ref, w1_ref, b1_ref, wmu_ref, atsmu_ref, atdmu_ref,
                  wls_ref, atsls_ref, atdls_ref,
                  hmu_ref, asmu_ref, admu_ref, hls_ref, asls_ref, adls_ref):
  w0 = w0_ref[0, 0]
  w1 = w1_ref[0, 0]
  inv = 1.0 / (w0 * s0_ref[...] + w1 * s1_ref[...] + 1e-16)
  n0 = (w0 * aa0_ref[...] + w1 * aa1_ref[...]) * inv
  n1 = (w0 * ab0_ref[...] + w1 * ab1_ref[...]) * inv
  z = jnp.concatenate([n0, n1], axis=1) + b1_ref[...]
  z = jnp.maximum(z, 0.0)
  hmu = jnp.dot(z, wmu_ref[...], preferred_element_type=_f32)
  hmu_ref[...] = hmu
  asmu_ref[...] = jnp.dot(hmu, atsmu_ref[...], preferred_element_type=_f32)
  admu_ref[...] = jnp.dot(hmu, atdmu_ref[...], preferred_element_type=_f32)
  hls = jnp.dot(z, wls_ref[...], preferred_element_type=_f32)
  hls_ref[...] = hls
  asls_ref[...] = jnp.dot(hls, atsls_ref[...], preferred_element_type=_f32)
  adls_ref[...] = jnp.dot(hls, atdls_ref[...], preferred_element_type=_f32)


def _combine(aa0, aa1, ab0, ab1, s0, s1, w0, w1, b1, wmu, atsmu, atdmu,
             wls, atsls, atdls):
  blk = lambda r, cdim: pl.BlockSpec((_BN, cdim), lambda i: (i, 0))
  cst = lambda a, b: pl.BlockSpec((a, b), lambda i: (0, 0))
  return pl.pallas_call(
      _combine_body,
      grid=(N // _BN,),
      in_specs=[
          blk(_BN, C), blk(_BN, C), blk(_BN, C), blk(_BN, C),
          blk(_BN, 1), blk(_BN, 1),
          cst(1, 1), cst(1, 1), cst(1, H),
          cst(H, C), cst(C, 1), cst(C, 1),
          cst(H, C), cst(C, 1), cst(C, 1),
      ],
      out_specs=[
          blk(_BN, C), blk(_BN, 1), blk(_BN, 1),
          blk(_BN, C), blk(_BN, 1), blk(_BN, 1),
      ],
      out_shape=[
          jax.ShapeDtypeStruct((N, C), _f32),
          jax.ShapeDtypeStruct((N, 1), _f32),
          jax.ShapeDtypeStruct((N, 1), _f32),
          jax.ShapeDtypeStruct((N, C), _f32),
          jax.ShapeDtypeStruct((N, 1), _f32),
          jax.ShapeDtypeStruct((N, 1), _f32),
      ],
  )(aa0, aa1, ab0, ab1, s0, s1, w0, w1, b1, wmu, atsmu, atdmu,
    wls, atsls, atdls)


def _finish_body(amu0_ref, amu1_ref, smu0_ref, smu1_ref, wmu0_ref, wmu1_ref,
                 bmu_ref, als0_ref, als1_ref, sls0_ref, sls1_ref,
                 wls0_ref, wls1_ref, bls_ref, mu_ref, ls_ref):
  w0 = wmu0_ref[0, 0]
  w1 = wmu1_ref[0, 0]
  inv = 1.0 / (w0 * smu0_ref[...] + w1 * smu1_ref[...] + 1e-16)
  mu_ref[...] = (w0 * amu0_ref[...] + w1 * amu1_ref[...]) * inv + bmu_ref[...]
  v0 = wls0_ref[0, 0]
  v1 = wls1_ref[0, 0]
  inv2 = 1.0 / (v0 * sls0_ref[...] + v1 * sls1_ref[...] + 1e-16)
  ls_ref[...] = (v0 * als0_ref[...] + v1 * als1_ref[...]) * inv2 + bls_ref[...]


def _finish(amu0, amu1, smu0, smu1, wmu0, wmu1, bmu,
            als0, als1, sls0, sls1, wls0, wls1, bls):
  blk = lambda cdim: pl.BlockSpec((_BN, cdim), lambda i: (i, 0))
  cst = lambda a, b: pl.BlockSpec((a, b), lambda i: (0, 0))
  return pl.pallas_call(
      _finish_body,
      grid=(N // _BN,),
      in_specs=[
          blk(C), blk(C), blk(1), blk(1), cst(1, 1), cst(1, 1), cst(1, C),
          blk(C), blk(C), blk(1), blk(1), cst(1, 1), cst(1, 1), cst(1, C),
      ],
      out_specs=[blk(C), blk(C)],
      out_shape=[
          jax.ShapeDtypeStruct((N, C), _f32),
          jax.ShapeDtypeStruct((N, C), _f32),
      ],
  )(amu0, amu1, smu0, smu1, wmu0, wmu1, bmu,
    als0, als1, sls0, sls1, wls0, wls1, bls)


def kernel(x, edge_index, edge_weights, W1, att_src1, att_dst1, bias1, We1,
           att_edge1, Wmu, att_src_mu, att_dst_mu, bias_mu, We_mu,
           att_edge_mu, Wls, att_src_ls, att_dst_ls, bias_ls):
  pad = ET_PAD - ET
  loops = jnp.arange(N, dtype=jnp.int32)
  zpad = jnp.zeros((pad,), jnp.int32)
  src = jnp.concatenate([edge_index[0], loops, zpad])
  dst = jnp.concatenate([edge_index[1], loops, zpad])
  neg = jnp.full((pad,), -1e30, _f32)

  et1_e, esum1 = _eterm(edge_weights, We1, att_edge1[:, None])
  fill1 = esum1[0, 0] / E
  et1 = jnp.concatenate([et1_e[:, 0], jnp.full((N,), fill1, _f32), neg])

  etmu_e, esummu = _eterm(edge_weights, We_mu, att_edge_mu[:, None])
  fillmu = esummu[0, 0] / E
  etmu = jnp.concatenate([etmu_e[:, 0], jnp.full((N,), fillmu, _f32), neg])

  etls = jnp.concatenate([jnp.zeros((ET,), _f32), neg])

  h1, as1, ad1 = _dense1(x, W1, att_src1[:, None], att_dst1[:, None])

  acc_a, s1v, m1 = _sc_edge(h1[:, :C], as1[:, 0], ad1[:, 0], et1, src, dst)
  acc_b, _, _ = _sc_edge(h1[:, C:], as1[:, 0], ad1[:, 0], et1, src, dst)

  g1 = jnp.maximum(m1[0, 0], m1[1, 0])
  w10 = jnp.exp(m1[0, 0] - g1)[None, None]
  w11 = jnp.exp(m1[1, 0] - g1)[None, None]

  hmu, asmu, admu, hls, asls, adls = _combine(
      acc_a[0], acc_a[1], acc_b[0], acc_b[1],
      s1v[0][:, None], s1v[1][:, None], w10, w11, bias1[None, :],
      Wmu, att_src_mu[:, None], att_dst_mu[:, None],
      Wls, att_src_ls[:, None], att_dst_ls[:, None])

  accmu, smu, mmu = _sc_edge(hmu, asmu[:, 0], admu[:, 0], etmu, src, dst)
  accls, sls, mls = _sc_edge(hls, asls[:, 0], adls[:, 0], etls, src, dst)

  gmu = jnp.maximum(mmu[0, 0], mmu[1, 0])
  wmu0 = jnp.exp(mmu[0, 0] - gmu)[None, None]
  wmu1 = jnp.exp(mmu[1, 0] - gmu)[None, None]
  gls = jnp.maximum(mls[0, 0], mls[1, 0])
  wls0 = jnp.exp(mls[0, 0] - gls)[None, None]
  wls1 = jnp.exp(mls[1, 0] - gls)[None, None]

  mu, logstd = _finish(
      accmu[0], accmu[1], smu[0][:, None], smu[1][:, None], wmu0, wmu1,
      bias_mu[None, :],
      accls[0], accls[1], sls[0][:, None], sls[1][:, None], wls0, wls1,
      bias_ls[None, :])
  return (mu, logstd)


# speedup vs baseline: 18.1356x; 1.0396x over previous
import functools

import jax
import jax.numpy as jnp
from jax import lax
from jax.experimental import pallas as pl
from jax.experimental.pallas import tpu as pltpu
from jax.experimental.pallas import tpu_sc as plsc

N = 10000
E = 320000
D = 128
H = 256
C = 128
ET = E + N
NW = 32
CHUNK = 10368
ET_PAD = NW * CHUNK
K = 64
NP = 10112
TROW = 632
ZR = 88

_f32 = jnp.float32
_i32 = jnp.int32


FH = 64


def _build_sc_edge():
  mesh = plsc.VectorSubcoreMesh(core_axis_name="c", subcore_axis_name="s",
                                num_cores=2, num_subcores=16)

  @functools.partial(
      pl.kernel,
      out_type=(
          jax.ShapeDtypeStruct((2, 2, NP, FH), _f32),
          jax.ShapeDtypeStruct((2 * NP,), _f32),
          jax.ShapeDtypeStruct((256,), _f32),
      ),
      mesh=mesh,
      compiler_params=pltpu.CompilerParams(needs_layout_passes=False,
                                           use_tc_tiling_on_sc=False),
      scratch_types=[
          pltpu.VMEM((NP,), _f32),
          pltpu.VMEM((NP,), _f32),
          pltpu.VMEM((CHUNK,), _i32),
          pltpu.VMEM((CHUNK,), _i32),
          pltpu.VMEM((CHUNK,), _f32),
          pltpu.VMEM((CHUNK,), _f32),
          pltpu.VMEM((K, FH), _f32),
          pltpu.VMEM((K, FH), _f32),
          pltpu.VMEM((K, FH), _f32),
          pltpu.VMEM((K, FH), _f32),
          pltpu.VMEM((K,), _i32),
          pltpu.VMEM((K,), _i32),
          pltpu.VMEM((K,), _i32),
          pltpu.VMEM((K,), _i32),
          pltpu.VMEM((2 * K,), _i32),
          pltpu.VMEM((ZR, FH), _f32),
          pltpu.VMEM((640,), _f32),
          pltpu.VMEM((128,), _f32),
          pltpu.VMEM((2048,), _f32),
          pltpu.VMEM_SHARED((NP, FH), _f32),
          pltpu.VMEM_SHARED((NP,), _f32),
          pltpu.VMEM_SHARED((2048,), _f32),
          pltpu.SemaphoreType.DMA,
          pltpu.SemaphoreType.DMA,
      ],
  )
  def sc_edge(h0_hbm, h1_hbm, as_hbm, ad_hbm, et_hbm, src_hbm, dst_hbm,
              acc_out, s_out, m_out,
              as_v, ad_v, src_v, dst_v, et_v, alpha_v,
              gbuf0, gbuf1, sbuf0, sbuf1, sidx0, sidx1, didx0, didx1,
              vidx, zbuf, zvec, mxr, maxbuf, acc_sh, svec_sh, maxsh,
              gsem0, gsem1):
    c = lax.axis_index("c")
    s = lax.axis_index("s")
    wid = s * 2 + c
    base = wid * CHUNK
    gbuf = (gbuf0, gbuf1)
    sbuf = (sbuf0, sbuf1)
    sidx = (sidx0, sidx1)
    didx = (didx0, didx1)
    gsem = (gsem0, gsem1)
    NCH = CHUNK // K

    pltpu.sync_copy(as_hbm, as_v)
    pltpu.sync_copy(ad_hbm, ad_v)
    pltpu.sync_copy(src_hbm.at[pl.ds(base, CHUNK)], src_v)
    pltpu.sync_copy(dst_hbm.at[pl.ds(base, CHUNK)], dst_v)
    pltpu.sync_copy(et_hbm.at[pl.ds(base, CHUNK)], et_v)

    zeros16 = jnp.zeros((16,), _f32)

    def zero_zbuf(r, carry):
      for f in range(FH // 16):
        zbuf[r, pl.ds(f * 16, 16)] = zeros16
      return carry

    lax.fori_loop(0, ZR, zero_zbuf, 0)
    for j in range(640 // 16):
      zvec[pl.ds(j * 16, 16)] = zeros16

    def zero_acc_slice():
      for kk in range(7):
        pltpu.sync_copy(zbuf, acc_sh.at[pl.ds(s * TROW + kk * ZR, ZR)])
      pltpu.sync_copy(zbuf.at[pl.ds(0, 16)],
                      acc_sh.at[pl.ds(s * TROW + 7 * ZR, 16)])

    zero_acc_slice()

    @pl.when(s == 0)
    def _zero_svec():
      for kk in range(15):
        pltpu.sync_copy(zvec, svec_sh.at[pl.ds(kk * 640, 640)])
      pltpu.sync_copy(zvec.at[pl.ds(0, 512)], svec_sh.at[pl.ds(9600, 512)])

    def pass_a(i, mx):
      sl = pl.ds(i * 16, 16)
      si = src_v[sl]
      di = dst_v[sl]
      a = (plsc.load_gather(as_v, [si]) + plsc.load_gather(ad_v, [di])
           + et_v[sl])
      a = jnp.where(a >= 0.0, a, 0.2 * a)
      alpha_v[sl] = a
      return jnp.maximum(mx, a)

    mx = lax.fori_loop(0, CHUNK // 16, pass_a,
                       jnp.full((16,), -3e38, _f32))
    for j in range(8):
      mxr[pl.ds(j * 16, 16)] = mx
    pltpu.sync_copy(mxr, maxsh.at[pl.ds(s * 128, 128)])
    plsc.subcore_barrier()
    pltpu.sync_copy(maxsh, maxbuf)

    def red_max(j, m):
      return jnp.maximum(m, maxbuf[pl.ds(j * 128, 16)])

    mall = lax.fori_loop(0, 16, red_max, jnp.full((16,), -3e38, _f32))
    mtot = jnp.max(mall)

    @pl.when(s == 0)
    def _write_m():
      mv16 = jnp.full((16,), mtot, _f32)
      for j in range(8):
        mxr[pl.ds(j * 16, 16)] = mv16
      pltpu.sync_copy(mxr, m_out.at[pl.ds(c * 128, 128)])

    def pass_b(i, carry):
      sl = pl.ds(i * 16, 16)
      alpha_v[sl] = jnp.exp(alpha_v[sl] - mtot)
      return carry

    lax.fori_loop(0, CHUNK // 16, pass_b, 0)

    for half, h_hbm in enumerate((h0_hbm, h1_hbm)):
      if half == 1:
        zero_acc_slice()
      plsc.subcore_barrier()

      for j in range(2):
        for q in range(K // 16):
          sidx[j][pl.ds(q * 16, 16)] = src_v[pl.ds(j * K + q * 16, 16)]
        pltpu.async_copy(h_hbm.at[sidx[j]], gbuf[j], gsem[j])

      def group(g, carry):
        for j in range(2):
          i = 2 * g + j
          b = i * K
          pltpu.make_async_copy(h_hbm.at[sidx[j]], gbuf[j], gsem[j]).wait()

          for q in range(K // 16):
            didx[j][pl.ds(q * 16, 16)] = dst_v[pl.ds(b + q * 16, 16)]

          def scale_row(r, cc):
            e_b = plsc.load_gather(alpha_v, [jnp.full((16,), b + r, _i32)])
            for f in range(FH // 16):
              sl2 = pl.ds(f * 16, 16)
              sbuf[j][r, sl2] = gbuf[j][r, sl2] * e_b
            return cc

          lax.fori_loop(0, K, scale_row, 0)

          @pl.when(i + 2 < NCH)
          def _next():
            for q in range(K // 16):
              sidx[j][pl.ds(q * 16, 16)] = src_v[pl.ds(b + 2 * K + q * 16,
                                                       16)]
            pltpu.async_copy(h_hbm.at[sidx[j]], gbuf[j], gsem[j])

          pltpu.sync_copy(sbuf[j], acc_sh.at[didx[j]], add=True)
          if half == 0 and j == 1:
            for q in range(2 * K // 16):
              vidx[pl.ds(q * 16, 16)] = dst_v[pl.ds(b - K + q * 16, 16)]
            pltpu.sync_copy(alpha_v.at[pl.ds(b - K, 2 * K)],
                            svec_sh.at[vidx], add=True)
        return carry

      lax.fori_loop(0, NCH // 2, group, 0)
      plsc.subcore_barrier()

      pltpu.sync_copy(acc_sh.at[pl.ds(s * TROW, TROW)],
                      acc_out.at[c, half, pl.ds(s * TROW, TROW)])

    @pl.when(s == 0)
    def _write_s():
      pltpu.sync_copy(svec_sh, s_out.at[pl.ds(c * NP, NP)])

  return sc_edge


_SC_EDGE_CACHE = {}


def _sc_half_call(h0, h1, asv, adv, et, src, dst):
  if "k" not in _SC_EDGE_CACHE:
    _SC_EDGE_CACHE["k"] = _build_sc_edge()
  pad = jnp.zeros((NP - N,), _f32)
  return _SC_EDGE_CACHE["k"](
      h0, h1, jnp.concatenate([asv, pad]), jnp.concatenate([adv, pad]),
      et, src, dst)


_BN = 1000


def _dense1_body(x_ref, w_ref, ats_ref, atd_ref, h_ref, as_ref, ad_ref):
  h = jnp.dot(x_ref[...], w_ref[...], preferred_element_type=_f32)
  h_ref[...] = h
  as_ref[...] = jnp.dot(h, ats_ref[...], preferred_element_type=_f32)
  ad_ref[...] = jnp.dot(h, atd_ref[...], preferred_element_type=_f32)


def _dense1(x, w, ats, atd):
  grid = (N // _BN,)
  return pl.pallas_call(
      _dense1_body,
      grid=grid,
      in_specs=[
          pl.BlockSpec((_BN, D), lambda i: (i, 0)),
          pl.BlockSpec((D, H), lambda i: (0, 0)),
          pl.BlockSpec((H, 1), lambda i: (0, 0)),
          pl.BlockSpec((H, 1), lambda i: (0, 0)),
      ],
      out_specs=[
          pl.BlockSpec((_BN, H), lambda i: (i, 0)),
          pl.BlockSpec((_BN, 1), lambda i: (i, 0)),
          pl.BlockSpec((_BN, 1), lambda i: (i, 0)),
      ],
      out_shape=[
          jax.ShapeDtypeStruct((N, H), _f32),
          jax.ShapeDtypeStruct((N, 1), _f32),
          jax.ShapeDtypeStruct((N, 1), _f32),
      ],
  )(x, w, ats, atd)


_BE = 8000


def _eterm2_body(ew_ref, we1_ref, ate1_ref, wemu_ref, atemu_ref,
                 et1_ref, sum1_ref, etmu_ref, summu_ref):
  i = pl.program_id(0)
  ew = ew_ref[...]
  w41 = jnp.dot(we1_ref[...], ate1_ref[...], preferred_element_type=_f32)
  et1 = jnp.dot(ew, w41, preferred_element_type=_f32)
  et1_ref[...] = et1
  w4m = jnp.dot(wemu_ref[...], atemu_ref[...], preferred_element_type=_f32)
  etm = jnp.dot(ew, w4m, preferred_element_type=_f32)
  etmu_ref[...] = etm

  @pl.when(i == 0)
  def _():
    sum1_ref[...] = jnp.zeros_like(sum1_ref)
    summu_ref[...] = jnp.zeros_like(summu_ref)

  sum1_ref[...] += jnp.sum(et1, axis=(0, 1), keepdims=True)
  summu_ref[...] += jnp.sum(etm, axis=(0, 1), keepdims=True)


def _eterm2(ew, we1, ate1, wemu, atemu):
  cst = lambda a, b: pl.BlockSpec((a, b), lambda i: (0, 0))
  return pl.pallas_call(
      _eterm2_body,
      grid=(E // _BE,),
      in_specs=[
          pl.BlockSpec((_BE, 4), lambda i: (i, 0)),
          cst(4, H), cst(H, 1), cst(4, C), cst(C, 1),
      ],
      out_specs=[
          pl.BlockSpec((_BE, 1), lambda i: (i, 0)),
          cst(1, 1),
          pl.BlockSpec((_BE, 1), lambda i: (i, 0)),
          cst(1, 1),
      ],
      out_shape=[
          jax.ShapeDtypeStruct((E, 1), _f32),
          jax.ShapeDtypeStruct((1, 1), _f32),
          jax.ShapeDtypeStruct((E, 1), _f32),
          jax.ShapeDtypeStruct((1, 1), _f32),
      ],
  )(ew, we1, ate1, wemu, atemu)


def _combine_body(acc1_ref, acc2_ref, sres_ref, w0_ref, w1_ref, b1_ref,
                  wmu_ref, atsmu_ref, atdmu_ref,
                  wls_ref, atsls_ref, atdls_ref,
                  hmu_ref, asmu_ref, admu_ref, hls_ref, asls_ref, adls_ref):
  w0 = w0_ref[0, 0]
  w1 = w1_ref[0, 0]
  a1 = acc1_ref[...]
  a2 = acc2_ref[...]
  n0 = jnp.concatenate([a1[0, 0], a1[0, 1], a2[0, 0], a2[0, 1]], axis=1)
  n1 = jnp.concatenate([a1[1, 0], a1[1, 1], a2[1, 0], a2[1, 1]], axis=1)
  s0 = sres_ref[0]
  s1 = sres_ref[1]
  inv = 1.0 / (w0 * s0 + w1 * s1 + 1e-16)
  z = (w0 * n0 + w1 * n1) * inv + b1_ref[...]
  z = jnp.maximum(z, 0.0)
  hmu = jnp.dot(z, wmu_ref[...], preferred_element_type=_f32)
  hmu_ref[...] = hmu
  asmu_ref[...] = jnp.dot(hmu, atsmu_ref[...], preferred_element_type=_f32)
  admu_ref[...] = jnp.dot(hmu, atdmu_ref[...], preferred_element_type=_f32)
  hls = jnp.dot(z, wls_ref[...], preferred_element_type=_f32)
  hls_ref[...] = hls
  asls_ref[...] = jnp.dot(hls, atsls_ref[...], preferred_element_type=_f32)
  adls_ref[...] = jnp.dot(hls, atdls_ref[...], preferred_element_type=_f32)


def _combine(acc1, acc2, sres, w0, w1, b1, wmu, atsmu, atdmu,
             wls, atsls, atdls):
  blk = lambda cdim: pl.BlockSpec((_BN, cdim), lambda i: (i, 0))
  cst = lambda a, b: pl.BlockSpec((a, b), lambda i: (0, 0))
  accspec = pl.BlockSpec((2, 2, _BN, FH), lambda i: (0, 0, i, 0))
  return pl.pallas_call(
      _combine_body,
      grid=(N // _BN,),
      in_specs=[
          accspec, accspec,
          pl.BlockSpec((2, _BN, 1), lambda i: (0, i, 0)),
          cst(1, 1), cst(1, 1), cst(1, H),
          cst(H, C), cst(C, 1), cst(C, 1),
          cst(H, C), cst(C, 1), cst(C, 1),
      ],
      out_specs=[
          blk(C), blk(1), blk(1),
          blk(C), blk(1), blk(1),
      ],
      out_shape=[
          jax.ShapeDtypeStruct((N, C), _f32),
          jax.ShapeDtypeStruct((N, 1), _f32),
          jax.ShapeDtypeStruct((N, 1), _f32),
          jax.ShapeDtypeStruct((N, C), _f32),
          jax.ShapeDtypeStruct((N, 1), _f32),
          jax.ShapeDtypeStruct((N, 1), _f32),
      ],
  )(acc1, acc2, sres, w0, w1, b1, wmu, atsmu, atdmu, wls, atsls, atdls)


def _finish_body(amu_ref, smu_ref, wmu0_ref, wmu1_ref, bmu_ref,
                 als_ref, sls_ref, wls0_ref, wls1_ref, bls_ref,
                 mu_ref, ls_ref):
  w0 = wmu0_ref[0, 0]
  w1 = wmu1_ref[0, 0]
  am = amu_ref[...]
  n0 = jnp.concatenate([am[0, 0], am[0, 1]], axis=1)
  n1 = jnp.concatenate([am[1, 0], am[1, 1]], axis=1)
  inv = 1.0 / (w0 * smu_ref[0] + w1 * smu_ref[1] + 1e-16)
  mu_ref[...] = (w0 * n0 + w1 * n1) * inv + bmu_ref[...]
  v0 = wls0_ref[0, 0]
  v1 = wls1_ref[0, 0]
  al = als_ref[...]
  m0 = jnp.concatenate([al[0, 0], al[0, 1]], axis=1)
  m1 = jnp.concatenate([al[1, 0], al[1, 1]], axis=1)
  inv2 = 1.0 / (v0 * sls_ref[0] + v1 * sls_ref[1] + 1e-16)
  ls_ref[...] = (v0 * m0 + v1 * m1) * inv2 + bls_ref[...]


def _finish(amu, smu, wmu0, wmu1, bmu, als, sls, wls0, wls1, bls):
  blk = lambda cdim: pl.BlockSpec((_BN, cdim), lambda i: (i, 0))
  cst = lambda a, b: pl.BlockSpec((a, b), lambda i: (0, 0))
  accspec = pl.BlockSpec((2, 2, _BN, FH), lambda i: (0, 0, i, 0))
  sspec = pl.BlockSpec((2, _BN, 1), lambda i: (0, i, 0))
  return pl.pallas_call(
      _finish_body,
      grid=(N // _BN,),
      in_specs=[
          accspec, sspec, cst(1, 1), cst(1, 1), cst(1, C),
          accspec, sspec, cst(1, 1), cst(1, 1), cst(1, C),
      ],
      out_specs=[blk(C), blk(C)],
      out_shape=[
          jax.ShapeDtypeStruct((N, C), _f32),
          jax.ShapeDtypeStruct((N, C), _f32),
      ],
  )(amu, smu, wmu0, wmu1, bmu, als, sls, wls0, wls1, bls)


def kernel(x, edge_index, edge_weights, W1, att_src1, att_dst1, bias1, We1,
           att_edge1, Wmu, att_src_mu, att_dst_mu, bias_mu, We_mu,
           att_edge_mu, Wls, att_src_ls, att_dst_ls, bias_ls):
  pad = ET_PAD - ET
  loops = jnp.arange(N, dtype=jnp.int32)
  zpad = jnp.zeros((pad,), jnp.int32)
  src = jnp.concatenate([edge_index[0], loops, zpad])
  dst = jnp.concatenate([edge_index[1], loops, zpad])
  neg = jnp.full((pad,), -1e30, _f32)

  et1_e, esum1, etmu_e, esummu = _eterm2(
      edge_weights, We1, att_edge1[:, None], We_mu, att_edge_mu[:, None])
  fill1 = esum1[0, 0] / E
  et1 = jnp.concatenate([et1_e[:, 0], jnp.full((N,), fill1, _f32), neg])
  fillmu = esummu[0, 0] / E
  etmu = jnp.concatenate([etmu_e[:, 0], jnp.full((N,), fillmu, _f32), neg])
  etls = jnp.concatenate([jnp.zeros((ET,), _f32), neg])

  h1, as1, ad1 = _dense1(x, W1, att_src1[:, None], att_dst1[:, None])

  acc_a, s1v, m1 = _sc_half_call(h1[:, 0:FH], h1[:, FH:2 * FH],
                                 as1[:, 0], ad1[:, 0], et1, src, dst)
  acc_b, _, _ = _sc_half_call(h1[:, 2 * FH:3 * FH], h1[:, 3 * FH:4 * FH],
                              as1[:, 0], ad1[:, 0], et1, src, dst)

  g1 = jnp.maximum(m1[0], m1[128])
  w10 = jnp.exp(m1[0] - g1)[None, None]
  w11 = jnp.exp(m1[128] - g1)[None, None]

  hmu, asmu, admu, hls, asls, adls = _combine(
      acc_a, acc_b, s1v.reshape(2, NP)[:, :, None], w10, w11,
      bias1[None, :],
      Wmu, att_src_mu[:, None], att_dst_mu[:, None],
      Wls, att_src_ls[:, None], att_dst_ls[:, None])

  accmu, smu, mmu = _sc_half_call(hmu[:, :FH], hmu[:, FH:],
                                  asmu[:, 0], admu[:, 0], etmu, src, dst)
  accls, sls, mls = _sc_half_call(hls[:, :FH], hls[:, FH:],
                                  asls[:, 0], adls[:, 0], etls, src, dst)

  gmu = jnp.maximum(mmu[0], mmu[128])
  wmu0 = jnp.exp(mmu[0] - gmu)[None, None]
  wmu1 = jnp.exp(mmu[128] - gmu)[None, None]
  gls = jnp.maximum(mls[0], mls[128])
  wls0 = jnp.exp(mls[0] - gls)[None, None]
  wls1 = jnp.exp(mls[128] - gls)[None, None]

  mu, logstd = _finish(
      accmu, smu.reshape(2, NP)[:, :, None], wmu0, wmu1, bias_mu[None, :],
      accls, sls.reshape(2, NP)[:, :, None], wls0, wls1, bias_ls[None, :])
  return (mu, logstd)
